# baseline (speedup 1.0000x reference)
import functools
import math

import jax
import jax.numpy as jnp
from jax import lax
from jax.experimental import pallas as pl
from jax.experimental.pallas import tpu as pltpu
from jax.experimental.pallas import tpu_sc as plsc

VOCAB = 100000
D_MODEL = 768
BATCH = 4
SEQ = 4096
MAX_PERIOD = 10000

LANES = 16
NUM_WORKERS = 32
PER_WORKER = (BATCH * SEQ) // NUM_WORKERS
CHUNK = 64
NUM_CHUNKS = PER_WORKER // CHUNK
VREGS_PER_ROW = D_MODEL // LANES


def _positional_table():
    half = D_MODEL // 2
    freqs = jnp.exp(
        -math.log(MAX_PERIOD) * jnp.arange(0, half, dtype=jnp.float32) / half
    )
    args = jnp.arange(SEQ, dtype=jnp.float32)[:, None] * freqs[None]
    return jnp.concatenate([jnp.cos(args), jnp.sin(args)], axis=-1)


def kernel(input_ids, token_embedding_table):
    pos = _positional_table()
    ids_flat = input_ids.reshape(BATCH * SEQ)

    info = plsc.get_sparse_core_info()
    num_cores = info.num_cores
    mesh = plsc.VectorSubcoreMesh(core_axis_name="c", subcore_axis_name="s")

    @functools.partial(
        pl.kernel,
        mesh=mesh,
        out_type=jax.ShapeDtypeStruct((BATCH * SEQ, D_MODEL), jnp.float32),
        scratch_types=[
            pltpu.VMEM((CHUNK,), jnp.int32),
            pltpu.VMEM((CHUNK, D_MODEL), jnp.float32),
            pltpu.VMEM((CHUNK, D_MODEL), jnp.float32),
            pltpu.SemaphoreType.DMA,
            pltpu.SemaphoreType.DMA,
        ],
    )
    def embed(ids_hbm, table_hbm, pos_hbm, out_hbm, idx_v, rows_v, pos_v, gsem, psem):
        wid = lax.axis_index("s") * num_cores + lax.axis_index("c")
        base = wid * PER_WORKER
        s_base = lax.rem(base, SEQ)

        def chunk_body(c, carry):
            off = base + c * CHUNK
            soff = s_base + c * CHUNK
            pltpu.sync_copy(ids_hbm.at[pl.ds(off, CHUNK)], idx_v)
            gather = pltpu.async_copy(table_hbm.at[idx_v], rows_v, gsem)
            poscp = pltpu.async_copy(pos_hbm.at[pl.ds(soff, CHUNK)], pos_v, psem)
            gather.wait()
            poscp.wait()

            def row_add(r, rcarry):
                for g in range(VREGS_PER_ROW):
                    sl = pl.ds(g * LANES, LANES)
                    plsc.addupdate(rows_v.at[r, sl], pos_v[r, sl])
                return rcarry

            lax.fori_loop(0, CHUNK, row_add, 0)
            pltpu.sync_copy(rows_v, out_hbm.at[pl.ds(off, CHUNK)])
            return carry

        lax.fori_loop(0, NUM_CHUNKS, chunk_body, 0)

    out = embed(ids_flat, token_embedding_table, pos)
    return out.reshape(BATCH, SEQ, D_MODEL)


# speedup vs baseline: 1.3546x; 1.3546x over previous
import functools
import math

import jax
import jax.numpy as jnp
from jax import lax
from jax.experimental import pallas as pl
from jax.experimental.pallas import tpu as pltpu
from jax.experimental.pallas import tpu_sc as plsc

VOCAB = 100000
D_MODEL = 768
BATCH = 4
SEQ = 4096
MAX_PERIOD = 10000

LANES = 16
NUM_WORKERS = 32
POS_PER_WORKER = SEQ // NUM_WORKERS
CHUNK = 32
NUM_CHUNKS = POS_PER_WORKER // CHUNK
NUM_STEPS = NUM_CHUNKS * BATCH
VREGS_PER_ROW = D_MODEL // LANES


def _positional_table():
    half = D_MODEL // 2
    freqs = jnp.exp(
        -math.log(MAX_PERIOD) * jnp.arange(0, half, dtype=jnp.float32) / half
    )
    args = jnp.arange(SEQ, dtype=jnp.float32)[:, None] * freqs[None]
    return jnp.concatenate([jnp.cos(args), jnp.sin(args)], axis=-1)


def kernel(input_ids, token_embedding_table):
    pos = _positional_table()

    info = plsc.get_sparse_core_info()
    num_cores = info.num_cores
    mesh = plsc.VectorSubcoreMesh(core_axis_name="c", subcore_axis_name="s")

    @functools.partial(
        pl.kernel,
        mesh=mesh,
        out_type=jax.ShapeDtypeStruct((BATCH * SEQ, D_MODEL), jnp.float32),
        scratch_types=[
            pltpu.VMEM((BATCH, POS_PER_WORKER), jnp.int32),
            pltpu.VMEM((CHUNK, D_MODEL), jnp.float32),
            pltpu.VMEM((CHUNK, D_MODEL), jnp.float32),
            pltpu.VMEM((CHUNK, D_MODEL), jnp.float32),
            pltpu.VMEM((CHUNK, D_MODEL), jnp.float32),
            pltpu.SemaphoreType.DMA,
            pltpu.SemaphoreType.DMA,
            pltpu.SemaphoreType.DMA,
            pltpu.SemaphoreType.DMA,
            pltpu.SemaphoreType.DMA,
            pltpu.SemaphoreType.DMA,
        ],
    )
    def embed(ids_hbm, table_hbm, pos_hbm, out_hbm,
              idx_all, rows0, rows1, pbuf0, pbuf1,
              g0, g1, p0, p1, w0, w1):
        wid = lax.axis_index("s") * num_cores + lax.axis_index("c")
        s0 = wid * POS_PER_WORKER
        rows = (rows0, rows1)
        pbuf = (pbuf0, pbuf1)
        gsem = (g0, g1)
        psem = (p0, p1)
        wsem = (w0, w1)

        for b in range(BATCH):
            pltpu.sync_copy(ids_hbm.at[b, pl.ds(s0, POS_PER_WORKER)],
                            idx_all.at[b])

        def issue_gather(t):
            c, b, j = t // BATCH, t % BATCH, t % 2
            return pltpu.async_copy(
                table_hbm.at[idx_all.at[b, pl.ds(c * CHUNK, CHUNK)]],
                rows[j], gsem[j])

        def issue_pos(c):
            return pltpu.async_copy(
                pos_hbm.at[pl.ds(s0 + c * CHUNK, CHUNK)],
                pbuf[c % 2], psem[c % 2])

        pcopies = [issue_pos(0), None]
        gcopies = [None] * NUM_STEPS
        wcopies = [None] * NUM_STEPS
        gcopies[0] = issue_gather(0)
        gcopies[1] = issue_gather(1)

        for t in range(NUM_STEPS):
            c, b, j = t // BATCH, t % BATCH, t % 2
            if b == 0:
                pcopies[c % 2].wait()
                if c + 1 < NUM_CHUNKS:
                    pcopies[(c + 1) % 2] = issue_pos(c + 1)
            if 1 <= t < NUM_STEPS - 1:
                wcopies[t - 1].wait()
                gcopies[t + 1] = issue_gather(t + 1)
            gcopies[t].wait()
            rv = rows[j]
            pv = pbuf[c % 2]

            def row_add(r, carry):
                for g in range(VREGS_PER_ROW):
                    sl = pl.ds(g * LANES, LANES)
                    plsc.addupdate(rv.at[r, sl], pv[r, sl])
                return carry

            lax.fori_loop(0, CHUNK, row_add, 0)
            wcopies[t] = pltpu.async_copy(
                rv, out_hbm.at[pl.ds(b * SEQ + s0 + c * CHUNK, CHUNK)], wsem[j])
        wcopies[NUM_STEPS - 2].wait()
        wcopies[NUM_STEPS - 1].wait()

    out = embed(input_ids, token_embedding_table, pos)
    return out.reshape(BATCH, SEQ, D_MODEL)


# speedup vs baseline: 1.7230x; 1.2720x over previous
import functools
import math

import jax
import jax.numpy as jnp
import numpy as np
from jax import lax
from jax.experimental import pallas as pl
from jax.experimental.pallas import tpu as pltpu
from jax.experimental.pallas import tpu_sc as plsc

VOCAB = 100000
D_MODEL = 768
BATCH = 4
SEQ = 4096
MAX_PERIOD = 10000

LANES = 16
NUM_WORKERS = 32
POS_PER_WORKER = SEQ // NUM_WORKERS
CHUNK = 32
NUM_CHUNKS = POS_PER_WORKER // CHUNK
NUM_STEPS = NUM_CHUNKS * BATCH
VREGS_PER_ROW = D_MODEL // LANES


def _positional_table():
    half = D_MODEL // 2
    freqs = np.exp(
        -math.log(MAX_PERIOD) * np.arange(0, half, dtype=np.float32) / half
    )
    args = np.arange(SEQ, dtype=np.float32)[:, None] * freqs[None]
    table = np.concatenate([np.cos(args), np.sin(args)], axis=-1)
    return table.astype(np.float32)


def kernel(input_ids, token_embedding_table):
    pos = _positional_table()

    info = plsc.get_sparse_core_info()
    num_cores = info.num_cores
    mesh = plsc.VectorSubcoreMesh(core_axis_name="c", subcore_axis_name="s")

    @functools.partial(
        pl.kernel,
        mesh=mesh,
        out_type=jax.ShapeDtypeStruct((BATCH * SEQ, D_MODEL), jnp.float32),
        scratch_types=[
            pltpu.VMEM((BATCH, POS_PER_WORKER), jnp.int32),
            pltpu.VMEM((CHUNK, D_MODEL), jnp.float32),
            pltpu.VMEM((CHUNK, D_MODEL), jnp.float32),
            pltpu.VMEM((CHUNK, D_MODEL), jnp.float32),
            pltpu.VMEM((CHUNK, D_MODEL), jnp.float32),
            pltpu.VMEM((CHUNK, D_MODEL), jnp.float32),
            pltpu.SemaphoreType.DMA,
            pltpu.SemaphoreType.DMA,
            pltpu.SemaphoreType.DMA,
            pltpu.SemaphoreType.DMA,
            pltpu.SemaphoreType.DMA,
            pltpu.SemaphoreType.DMA,
            pltpu.SemaphoreType.DMA,
            pltpu.SemaphoreType.DMA,
        ],
    )
    def embed(ids_hbm, table_hbm, pos_hbm, out_hbm,
              idx_all, rows0, rows1, rows2, pbuf0, pbuf1,
              g0, g1, g2, p0, p1, w0, w1, w2):
        wid = lax.axis_index("s") * num_cores + lax.axis_index("c")
        s0 = wid * POS_PER_WORKER
        rows = (rows0, rows1, rows2)
        pbuf = (pbuf0, pbuf1)
        gsem = (g0, g1, g2)
        psem = (p0, p1)
        wsem = (w0, w1, w2)

        for b in range(BATCH):
            pltpu.sync_copy(ids_hbm.at[b, pl.ds(s0, POS_PER_WORKER)],
                            idx_all.at[b])

        def issue_gather(t):
            c, b, j = t // BATCH, t % BATCH, t % 3
            return pltpu.async_copy(
                table_hbm.at[idx_all.at[b, pl.ds(c * CHUNK, CHUNK)]],
                rows[j], gsem[j])

        def issue_pos(c):
            return pltpu.async_copy(
                pos_hbm.at[pl.ds(s0 + c * CHUNK, CHUNK)],
                pbuf[c % 2], psem[c % 2])

        pcopies = [issue_pos(0), None]
        gcopies = [None] * NUM_STEPS
        wcopies = [None] * NUM_STEPS
        gcopies[0] = issue_gather(0)
        gcopies[1] = issue_gather(1)

        for t in range(NUM_STEPS):
            c, b, j = t // BATCH, t % BATCH, t % 3
            if b == 0:
                pcopies[c % 2].wait()
                if c + 1 < NUM_CHUNKS:
                    pcopies[(c + 1) % 2] = issue_pos(c + 1)
            gcopies[t].wait()
            rv = rows[j]
            pv = pbuf[c % 2]

            def row_add(r, carry):
                for g in range(VREGS_PER_ROW):
                    sl = pl.ds(g * LANES, LANES)
                    plsc.addupdate(rv.at[r, sl], pv[r, sl])
                return carry

            lax.fori_loop(0, CHUNK, row_add, 0)
            wcopies[t] = pltpu.async_copy(
                rv, out_hbm.at[pl.ds(b * SEQ + s0 + c * CHUNK, CHUNK)], wsem[j])
            if t + 2 < NUM_STEPS:
                if t >= 1:
                    wcopies[t - 1].wait()
                gcopies[t + 2] = issue_gather(t + 2)
        wcopies[NUM_STEPS - 3].wait()
        wcopies[NUM_STEPS - 2].wait()
        wcopies[NUM_STEPS - 1].wait()

    out = embed(input_ids, token_embedding_table, pos)
    return out.reshape(BATCH, SEQ, D_MODEL)
